# Initial kernel scaffold; baseline (speedup 1.0000x reference)
import jax
import jax.numpy as jnp
from jax.experimental import pallas as pl


def kernel(x, pos_table):
    raise NotImplementedError("write your pallas kernel here")


# speedup vs baseline: 1.6622x; 1.6622x over previous
import jax
import jax.numpy as jnp
from jax.experimental import pallas as pl

BLOCK_S = 512


def _add_kernel(x_ref, pos_ref, o_ref):
    o_ref[...] = x_ref[...] + pos_ref[...]


def kernel(x, pos_table):
    batch, seq_len, embed_dim = x.shape
    pos = pos_table[:seq_len]
    num_s = seq_len // BLOCK_S
    return pl.pallas_call(
        _add_kernel,
        grid=(num_s, batch),
        in_specs=[
            pl.BlockSpec((1, BLOCK_S, embed_dim), lambda i, j: (j, i, 0)),
            pl.BlockSpec((BLOCK_S, embed_dim), lambda i, j: (i, 0)),
        ],
        out_specs=pl.BlockSpec((1, BLOCK_S, embed_dim), lambda i, j: (j, i, 0)),
        out_shape=jax.ShapeDtypeStruct(x.shape, x.dtype),
    )(x, pos)


# speedup vs baseline: 1.7350x; 1.0438x over previous
import jax
import jax.numpy as jnp
from jax.experimental import pallas as pl

BLOCK_S = 1024


def _add_kernel(x_ref, pos_ref, o_ref):
    o_ref[...] = x_ref[...] + pos_ref[...]


def kernel(x, pos_table):
    batch, seq_len, embed_dim = x.shape
    pos = pos_table[:seq_len]
    num_s = seq_len // BLOCK_S
    return pl.pallas_call(
        _add_kernel,
        grid=(num_s, batch),
        in_specs=[
            pl.BlockSpec((1, BLOCK_S, embed_dim), lambda i, j: (j, i, 0)),
            pl.BlockSpec((BLOCK_S, embed_dim), lambda i, j: (i, 0)),
        ],
        out_specs=pl.BlockSpec((1, BLOCK_S, embed_dim), lambda i, j: (j, i, 0)),
        out_shape=jax.ShapeDtypeStruct(x.shape, x.dtype),
    )(x, pos)
